# Initial kernel scaffold; baseline (speedup 1.0000x reference)
import jax
import jax.numpy as jnp
from jax.experimental import pallas as pl


def kernel(occupy, level, octant, laser, phi, pos, E0, E1, E2, E3, E4, W_pos, W_fuse, b_fuse, Wq, Wk, Wv, Wo):
    raise NotImplementedError("write your pallas kernel here")


# speedup vs baseline: 5.1880x; 5.1880x over previous
import jax
import jax.numpy as jnp
from jax.experimental import pallas as pl
from jax.experimental.pallas import tpu as pltpu

S, B, CTX, K_NN, D = 2048, 4, 4, 16, 256
N = S * B
T1 = 256
R = 256
HI = jax.lax.Precision.HIGHEST


def _norm_body(pos_ref, out_ref):
    p = pos_ref[...]
    pmin = jnp.min(p, axis=0, keepdims=True)
    pmax = jnp.max(p, axis=0, keepdims=True)
    out_ref[...] = (p - pmin) / (pmax - pmin + 1e-07)


def _embed_body(occ_ref, lev_ref, oct_ref, las_ref, phi_ref, pn3_ref,
                E0_ref, E1_ref, E2_ref, E3_ref, E4_ref,
                Wp_ref, Wf_ref, bf_ref, Wq_ref, Wk_ref, Wv_ref,
                e_ref, q_ref, k_ref, v_ref, pn_ref):
    def emb(idx_col, table_ref, vocab):
        oh = (idx_col[:, None] ==
              jax.lax.broadcasted_iota(jnp.int32, (T1, vocab), 1))
        return jax.lax.dot(oh.astype(jnp.float32), table_ref[...],
                           precision=HI)

    parts = []
    for c in range(CTX):
        parts.append(emb(occ_ref[:, c], E0_ref, 256))
        parts.append(emb(lev_ref[:, c], E1_ref, 16))
        parts.append(emb(oct_ref[:, c], E2_ref, 8))
    for c in range(CTX):
        parts.append(emb(las_ref[:, c], E3_ref, 32))
        parts.append(emb(phi_ref[:, c], E4_ref, 2250))
    pn = jax.lax.dot(pn3_ref[...], Wp_ref[...], precision=HI)
    parts.append(pn)
    full = jnp.concatenate(parts, axis=-1)
    e = jax.lax.dot(full, Wf_ref[...], precision=HI) + bf_ref[...][None, :]
    e_ref[...] = e
    q_ref[...] = jax.lax.dot(e, Wq_ref[...], precision=HI)
    k_ref[...] = jax.lax.dot(e, Wk_ref[...], precision=HI)
    v_ref[...] = jax.lax.dot(e, Wv_ref[...], precision=HI)
    pn_ref[...] = pn


def _attn_body(pnr_ref, pnf_ref, q_ref, k_ref, v_ref, e_ref, Wo_ref,
               out_ref):
    pnr = pnr_ref[0]
    pnf = pnf_ref[0]
    sqr = jnp.sum(pnr * pnr, axis=1)
    sqf = jnp.sum(pnf * pnf, axis=1)
    cross = jax.lax.dot_general(pnr, pnf, (((1,), (1,)), ((), ())),
                                precision=HI)
    d = sqr[:, None] + sqf[None, :] - 2.0 * cross
    col = jax.lax.broadcasted_iota(jnp.int32, (R, S), 1)
    sel = jnp.zeros((R, S), dtype=jnp.bool_)
    for _ in range(K_NN):
        m = jnp.min(d, axis=1)
        eq = d == m[:, None]
        j = jnp.min(jnp.where(eq, col, S), axis=1)
        hit = col == j[:, None]
        sel = jnp.logical_or(sel, hit)
        d = jnp.where(hit, jnp.float32(jnp.inf), d)
    logits = jax.lax.dot_general(q_ref[0], k_ref[0],
                                 (((1,), (1,)), ((), ())),
                                 precision=HI) * (1.0 / 16.0)
    ml = jnp.where(sel, logits, -jnp.inf)
    mmax = jnp.max(ml, axis=1)
    p = jnp.exp(ml - mmax[:, None])
    att = p / jnp.sum(p, axis=1)[:, None]
    o = jax.lax.dot(att, v_ref[0], precision=HI)
    out_ref[0] = jax.lax.dot(o, Wo_ref[...], precision=HI) + e_ref[0]


def _full(shape):
    nd = len(shape)
    return pl.BlockSpec(shape, lambda *a, s=nd: (0,) * s)


@jax.jit
def kernel(occupy, level, octant, laser, phi, pos,
           E0, E1, E2, E3, E4, W_pos, W_fuse, b_fuse, Wq, Wk, Wv, Wo):
    occ = occupy.reshape(N, CTX).astype(jnp.int32)
    lev = level.reshape(N, CTX).astype(jnp.int32)
    oct_ = octant.reshape(N, CTX).astype(jnp.int32)
    las = laser.reshape(N, CTX).astype(jnp.int32)
    ph = phi.reshape(N, CTX).astype(jnp.int32)

    pn3 = pl.pallas_call(
        _norm_body,
        out_shape=jax.ShapeDtypeStruct((S, B, 3), jnp.float32),
    )(pos)
    pn3 = pn3.reshape(N, 3)

    tok = lambda w: pl.BlockSpec((T1, w), lambda i: (i, 0))
    e, q, k, v, pn = pl.pallas_call(
        _embed_body,
        grid=(N // T1,),
        in_specs=[tok(CTX)] * 5 + [tok(3)] + [
            _full(E0.shape), _full(E1.shape), _full(E2.shape),
            _full(E3.shape), _full(E4.shape),
            _full(W_pos.shape), _full(W_fuse.shape),
            pl.BlockSpec((256,), lambda i: (0,)),
            _full(Wq.shape), _full(Wk.shape), _full(Wv.shape),
        ],
        out_specs=[tok(256), tok(256), tok(256), tok(256), tok(128)],
        out_shape=[jax.ShapeDtypeStruct((N, 256), jnp.float32)] * 4
        + [jax.ShapeDtypeStruct((N, 128), jnp.float32)],
    )(occ, lev, oct_, las, ph, pn3,
      E0, E1, E2, E3, E4, W_pos, W_fuse, b_fuse, Wq, Wk, Wv)

    def to_bs(x):
        return x.reshape(S, B, -1).transpose(1, 0, 2)

    pn_t, e_t, q_t, k_t, v_t = map(to_bs, (pn, e, q, k, v))

    row = lambda w: pl.BlockSpec((1, R, w), lambda b, r: (b, r, 0))
    allrows = lambda w: pl.BlockSpec((1, S, w), lambda b, r: (b, 0, 0))
    out = pl.pallas_call(
        _attn_body,
        grid=(B, S // R),
        in_specs=[row(128), allrows(128), row(256), allrows(256),
                  allrows(256), row(256), _full(Wo.shape)],
        out_specs=row(256),
        out_shape=jax.ShapeDtypeStruct((B, S, 256), jnp.float32),
    )(pn_t, pn_t, q_t, k_t, v_t, e_t, Wo)
    return out


# speedup vs baseline: 13.9976x; 2.6981x over previous
import jax
import jax.numpy as jnp
from jax.experimental import pallas as pl
from jax.experimental.pallas import tpu as pltpu

S, B, CTX, K_NN, D = 2048, 4, 4, 16, 256
N = S * B
T1 = 256
R = 256


def _norm_body(pos_ref, out_ref):
    p = pos_ref[...]
    pmin = jnp.min(p, axis=0, keepdims=True)
    pmax = jnp.max(p, axis=0, keepdims=True)
    out_ref[...] = (p - pmin) / (pmax - pmin + 1e-07)


def _embed_body(occ_ref, lev_ref, oct_ref, las_ref, phi_ref, pn3_ref,
                E0_ref, E1_ref, E2_ref, E3_ref, E4_ref,
                Wp_ref, Wf_ref, bf_ref, Wq_ref, Wk_ref, Wv_ref,
                e_ref, q_ref, k_ref, v_ref, pn_ref):
    def emb(idx_col, table_ref, vocab):
        oh = (idx_col[:, None] ==
              jax.lax.broadcasted_iota(jnp.int32, (T1, vocab), 1))
        return jax.lax.dot(oh.astype(jnp.float32), table_ref[...])

    parts = []
    for c in range(CTX):
        parts.append(emb(occ_ref[:, c], E0_ref, 256))
        parts.append(emb(lev_ref[:, c], E1_ref, 16))
        parts.append(emb(oct_ref[:, c], E2_ref, 8))
    for c in range(CTX):
        parts.append(emb(las_ref[:, c], E3_ref, 32))
        parts.append(emb(phi_ref[:, c], E4_ref, 2250))
    pn = jax.lax.dot(pn3_ref[...], Wp_ref[...])
    parts.append(pn)
    full = jnp.concatenate(parts, axis=-1)
    e = jax.lax.dot(full, Wf_ref[...]) + bf_ref[...][None, :]
    e_ref[...] = e
    q_ref[...] = jax.lax.dot(e, Wq_ref[...])
    k_ref[...] = jax.lax.dot(e, Wk_ref[...])
    v_ref[...] = jax.lax.dot(e, Wv_ref[...])
    pn_ref[...] = pn


def _attn_body(pnr_ref, pnf_ref, q_ref, k_ref, v_ref, e_ref, Wo_ref,
               out_ref):
    pnr = pnr_ref[0]
    pnf = pnf_ref[0]
    sqr = jnp.sum(pnr * pnr, axis=1)
    sqf = jnp.sum(pnf * pnf, axis=1)
    cross = jax.lax.dot_general(pnr, pnf, (((1,), (1,)), ((), ())))
    d = sqr[:, None] + sqf[None, :] - 2.0 * cross
    sel = jnp.zeros((R, S), dtype=jnp.bool_)
    for _ in range(K_NN):
        m = jnp.min(d, axis=1)
        hit = d == m[:, None]
        sel = jnp.logical_or(sel, hit)
        d = jnp.where(hit, jnp.float32(jnp.inf), d)
    logits = jax.lax.dot_general(q_ref[0], k_ref[0],
                                 (((1,), (1,)), ((), ()))) * (1.0 / 16.0)
    ml = jnp.where(sel, logits, -jnp.inf)
    mmax = jnp.max(ml, axis=1)
    p = jnp.exp(ml - mmax[:, None])
    att = p / jnp.sum(p, axis=1)[:, None]
    o = jax.lax.dot(att, v_ref[0])
    out_ref[0] = jax.lax.dot(o, Wo_ref[...]) + e_ref[0]


def _full(shape):
    nd = len(shape)
    return pl.BlockSpec(shape, lambda *a, s=nd: (0,) * s)


@jax.jit
def kernel(occupy, level, octant, laser, phi, pos,
           E0, E1, E2, E3, E4, W_pos, W_fuse, b_fuse, Wq, Wk, Wv, Wo):
    def bm(x):
        return x.astype(jnp.int32).transpose(1, 0, 2).reshape(N, CTX)

    occ, lev, oct_, las, ph = map(bm, (occupy, level, octant, laser, phi))

    pn3 = pl.pallas_call(
        _norm_body,
        out_shape=jax.ShapeDtypeStruct((S, B, 3), jnp.float32),
    )(pos)
    pn3 = pn3.transpose(1, 0, 2).reshape(N, 3)

    tok = lambda w: pl.BlockSpec((T1, w), lambda i: (i, 0))
    e, q, k, v, pn = pl.pallas_call(
        _embed_body,
        grid=(N // T1,),
        in_specs=[tok(CTX)] * 5 + [tok(3)] + [
            _full(E0.shape), _full(E1.shape), _full(E2.shape),
            _full(E3.shape), _full(E4.shape),
            _full(W_pos.shape), _full(W_fuse.shape),
            pl.BlockSpec((256,), lambda i: (0,)),
            _full(Wq.shape), _full(Wk.shape), _full(Wv.shape),
        ],
        out_specs=[tok(256), tok(256), tok(256), tok(256), tok(128)],
        out_shape=[jax.ShapeDtypeStruct((N, 256), jnp.float32)] * 4
        + [jax.ShapeDtypeStruct((N, 128), jnp.float32)],
    )(occ, lev, oct_, las, ph, pn3,
      E0, E1, E2, E3, E4, W_pos, W_fuse, b_fuse, Wq, Wk, Wv)

    pn_t = pn.reshape(B, S, 128)
    e_t, q_t, k_t, v_t = (x.reshape(B, S, 256) for x in (e, q, k, v))

    row = lambda w: pl.BlockSpec((1, R, w), lambda b, r: (b, r, 0))
    allrows = lambda w: pl.BlockSpec((1, S, w), lambda b, r: (b, 0, 0))
    out = pl.pallas_call(
        _attn_body,
        grid=(B, S // R),
        in_specs=[row(128), allrows(128), row(256), allrows(256),
                  allrows(256), row(256), _full(Wo.shape)],
        out_specs=row(256),
        out_shape=jax.ShapeDtypeStruct((B, S, 256), jnp.float32),
    )(pn_t, pn_t, q_t, k_t, v_t, e_t, Wo)
    return out
